# baseline (speedup 1.0000x reference)
import functools

import jax
import jax.numpy as jnp
from jax import lax
from jax.experimental import pallas as pl
from jax.experimental.pallas import tpu as pltpu
from jax.experimental.pallas import tpu_sc as plsc

_MAX_VALUE = 1000.0
_ORD = 20.0
_CUT = 0.1
_N = 50000
_S = 100000
_A = 16
_E = _N * _A
_ITERS = 8
_BN = 4096

_NW = 32
_PW = _E // _NW
_CH = 25008
_PSUB = (6256, 6256, 6256, 6240)


def _worker_start(wid):
    return wid * _PW - 8 * (wid % 2)


def _wid():
    return lax.axis_index("s") * 2 + lax.axis_index("c")


def _gather_loop(table_v, idx_v, out_v, n_vec):
    @plsc.parallel_loop(0, n_vec * 16, step=16, unroll=16)
    def _(i):
        idx = idx_v[pl.ds(i, 16)]
        out_v[pl.ds(i, 16)] = plsc.load_gather(table_v, [idx])


_CH3 = _CH // 3


def _iter_gather_body(u_hbm, jk_hbm, uj_hbm, uk_hbm, u_v, idx0, idx1, outj0,
                      outj1, outk0, outk1, u_sem, isem0, isem1, ojsem0,
                      ojsem1, oksem0, oksem1):
    start = _worker_start(_wid())
    idx_bufs = (idx0, idx1)
    outj_bufs = (outj0, outj1)
    outk_bufs = (outk0, outk1)
    isems = (isem0, isem1)
    ojsems = (ojsem0, ojsem1)
    oksems = (oksem0, oksem1)
    u_copy = pltpu.make_async_copy(u_hbm, u_v, u_sem)
    u_copy.start()

    def in_copy(c):
        return pltpu.make_async_copy(
            jk_hbm.at[pl.ds(start + c * _CH3, _CH3)], idx_bufs[c % 2],
            isems[c % 2])

    def out_copies(c):
        return (
            pltpu.make_async_copy(
                outj_bufs[c % 2], uj_hbm.at[pl.ds(start + c * _CH3, _CH3)],
                ojsems[c % 2]),
            pltpu.make_async_copy(
                outk_bufs[c % 2], uk_hbm.at[pl.ds(start + c * _CH3, _CH3)],
                oksems[c % 2]),
        )

    copies_in = [in_copy(c) for c in range(3)]
    copies_out = [out_copies(c) for c in range(3)]
    copies_in[0].start()
    for c in range(3):
        if c + 1 < 3:
            copies_in[c + 1].start()
        copies_in[c].wait()
        if c == 0:
            u_copy.wait()
        if c >= 2:
            for cp in copies_out[c - 2]:
                cp.wait()
        idx_v = idx_bufs[c % 2]
        outj_v = outj_bufs[c % 2]
        outk_v = outk_bufs[c % 2]

        @plsc.parallel_loop(0, _CH3, step=16, unroll=16)
        def _(i):
            jk = idx_v[pl.ds(i, 16)]
            jdx = jnp.bitwise_and(jk, 0xFFFF)
            kdx = lax.shift_right_logical(jk, 16)
            outj_v[pl.ds(i, 16)] = plsc.load_gather(u_v, [jdx])
            outk_v[pl.ds(i, 16)] = plsc.load_gather(u_v, [kdx])

        for cp in copies_out[c]:
            cp.start()
    for c in (1, 2):
        for cp in copies_out[c]:
            cp.wait()


_SC_PARAMS = pltpu.CompilerParams(needs_layout_passes=False)

_iter_gather = functools.partial(
    pl.kernel,
    mesh=plsc.VectorSubcoreMesh(core_axis_name="c", subcore_axis_name="s"),
    compiler_params=_SC_PARAMS,
    out_type=[jax.ShapeDtypeStruct((_E,), jnp.float32)] * 2,
    scratch_types=[
        pltpu.VMEM((_N,), jnp.float32),
        pltpu.VMEM((_CH3,), jnp.int32),
        pltpu.VMEM((_CH3,), jnp.int32),
        pltpu.VMEM((_CH3,), jnp.float32),
        pltpu.VMEM((_CH3,), jnp.float32),
        pltpu.VMEM((_CH3,), jnp.float32),
        pltpu.VMEM((_CH3,), jnp.float32),
        pltpu.SemaphoreType.DMA,
        pltpu.SemaphoreType.DMA,
        pltpu.SemaphoreType.DMA,
        pltpu.SemaphoreType.DMA,
        pltpu.SemaphoreType.DMA,
        pltpu.SemaphoreType.DMA,
        pltpu.SemaphoreType.DMA,
    ],
)(_iter_gather_body)


def _pre_gather_body(vx_hbm, vy_hbm, m00_hbm, m01_hbm, m11_hbm, jk_hbm,
                     sf_hbm, xjx_hbm, xjy_hbm, xkx_hbm, xky_hbm, g00_hbm,
                     g01_hbm, g11_hbm, table_v, idx_v, outa_v, outb_v):
    start = _worker_start(_wid())

    def vertex_chunk(outj_hbm, outk_hbm, off, cnt):
        pltpu.sync_copy(jk_hbm.at[pl.ds(off, cnt)], idx_v.at[pl.ds(0, cnt)])

        @plsc.parallel_loop(0, cnt, step=16, unroll=16)
        def _(i):
            jk = idx_v[pl.ds(i, 16)]
            jdx = jnp.bitwise_and(jk, 0xFFFF)
            kdx = lax.shift_right_logical(jk, 16)
            outa_v[pl.ds(i, 16)] = plsc.load_gather(table_v, [jdx])
            outb_v[pl.ds(i, 16)] = plsc.load_gather(table_v, [kdx])

        pltpu.sync_copy(outa_v.at[pl.ds(0, cnt)], outj_hbm.at[pl.ds(off,
                                                                    cnt)])
        pltpu.sync_copy(outb_v.at[pl.ds(0, cnt)], outk_hbm.at[pl.ds(off,
                                                                    cnt)])

    def sf_chunk(out_hbm, off, cnt):
        pltpu.sync_copy(sf_hbm.at[pl.ds(off, cnt)], idx_v.at[pl.ds(0, cnt)])
        _gather_loop(table_v, idx_v, outa_v, cnt // 16)
        pltpu.sync_copy(outa_v.at[pl.ds(0, cnt)], out_hbm.at[pl.ds(off,
                                                                   cnt)])

    offs = []
    off = 0
    for cnt in _PSUB:
        offs.append((off, cnt))
        off += cnt

    for tbl_hbm, outj_hbm, outk_hbm in ((vx_hbm, xjx_hbm, xkx_hbm),
                                        (vy_hbm, xjy_hbm, xky_hbm)):
        pltpu.sync_copy(tbl_hbm, table_v.at[pl.ds(0, _N)])
        for off, cnt in offs:
            vertex_chunk(outj_hbm, outk_hbm, start + off, cnt)

    for tbl_hbm, out_hbm in ((m00_hbm, g00_hbm), (m01_hbm, g01_hbm),
                             (m11_hbm, g11_hbm)):
        pltpu.sync_copy(tbl_hbm, table_v.at[pl.ds(0, _S)])
        for off, cnt in offs:
            sf_chunk(out_hbm, start + off, cnt)


_pre_gather = functools.partial(
    pl.kernel,
    mesh=plsc.VectorSubcoreMesh(core_axis_name="c", subcore_axis_name="s"),
    compiler_params=_SC_PARAMS,
    out_type=[jax.ShapeDtypeStruct((_E,), jnp.float32)] * 7,
    scratch_types=[
        pltpu.VMEM((_S,), jnp.float32),
        pltpu.VMEM((_PSUB[0],), jnp.int32),
        pltpu.VMEM((_PSUB[0],), jnp.float32),
        pltpu.VMEM((_PSUB[0],), jnp.float32),
    ],
)(_pre_gather_body)


def _softplus(x):
    return jnp.maximum(x, 0.0) + jnp.log1p(jnp.exp(-jnp.abs(x)))


def _drelu(x):
    return _softplus(_ORD * x) / _ORD - _softplus(_ORD * (x - 1.0)) / _ORD


def _geom_body(xjx_ref, xjy_ref, xkx_ref, xky_ref, g00_ref, g01_ref,
               g11_ref, vx_ref, vy_ref, eme_ref, ema_ref, ama_ref):
    xkx = xkx_ref[...]
    xky = xky_ref[...]
    g00 = g00_ref[...]
    g01 = g01_ref[...]
    g11 = g11_ref[...]
    ex = xjx_ref[...] - xkx
    ey = xjy_ref[...] - xky
    ax = vx_ref[...][None, :] - xkx
    ay = vy_ref[...][None, :] - xky
    mex = g00 * ex + g01 * ey
    mey = g01 * ex + g11 * ey
    max_ = g00 * ax + g01 * ay
    may_ = g01 * ax + g11 * ay
    eme_ref[...] = ex * mex + ey * mey
    ema_ref[...] = ex * max_ + ey * may_
    ama_ref[...] = ax * max_ + ay * may_


def _tc_geom(xjx, xjy, xkx, xky, g00, g01, g11, vx, vy):
    nblk = pl.cdiv(_N, _BN)
    spec_a = pl.BlockSpec((_A, _BN), lambda i: (0, i))
    spec_n = pl.BlockSpec((_BN,), lambda i: (i,))
    return pl.pallas_call(
        _geom_body,
        grid=(nblk,),
        in_specs=[spec_a] * 7 + [spec_n] * 2,
        out_specs=[spec_a] * 3,
        out_shape=[jax.ShapeDtypeStruct((_A, _N), jnp.float32)] * 3,
    )(xjx, xjy, xkx, xky, g00, g01, g11, vx, vy)


def _update_body(eme_ref, ema_ref, ama_ref, uj_ref, uk_ref, u_ref, mask_ref,
                 val_ref, out_ref):
    eme = eme_ref[...]
    ema = ema_ref[...]
    ama = ama_ref[...]
    uj = uj_ref[...]
    uk = uk_ref[...]

    delta = uj - uk
    d2 = delta * delta
    qa = eme * (eme - d2)
    qb = 2.0 * ema * (d2 - eme)
    qc = ema * ema - d2 * ama
    disc = qb * qb - 4.0 * qa * qc
    valid = disc >= 0.0
    sq = jnp.sqrt(jnp.maximum(disc, 0.0))
    denom = 2.0 * qa + jnp.where(jnp.abs(qa) < 1e-8, 1e-8, 0.0)
    lam1 = (-qb + sq) / denom
    lam2 = (-qb - sq) / denom
    l1c = _drelu(lam1)
    l2c = _drelu(lam2)

    def ttime(lam):
        dmd = ama + lam * (lam * eme - 2.0 * ema)
        return (1.0 - lam) * uk + lam * uj + jnp.sqrt(
            jnp.clip(dmd, 1e-8, None))

    t1 = jnp.where(valid & (jnp.abs(lam1 - l1c) <= _CUT), ttime(l1c),
                   _MAX_VALUE)
    t2 = jnp.where(valid & (jnp.abs(lam2 - l2c) <= _CUT), ttime(l2c),
                   _MAX_VALUE)
    t0 = uk + jnp.sqrt(jnp.clip(ama, 1e-8, None))
    t3 = uj + jnp.sqrt(jnp.clip(ama - 2.0 * ema + eme, 1e-8, None))
    o = _ORD
    m = jnp.maximum(jnp.maximum(-o * t1, -o * t2),
                    jnp.maximum(-o * t0, -o * t3))
    m2 = jnp.max(m, axis=0)
    mb = m2[None, :]
    ssum = ((jnp.exp(-o * t1 - mb) + jnp.exp(-o * t2 - mb)) +
            (jnp.exp(-o * t0 - mb) + jnp.exp(-o * t3 - mb)))
    s2 = jnp.sum(ssum, axis=0)
    pv = -(m2 + jnp.log(s2)) / o
    unew = jnp.minimum(pv, u_ref[...])
    out_ref[...] = jnp.where(mask_ref[...] > 0.5, unew, val_ref[...])


def _tc_update(eme, ema, ama, uj, uk, u, mask, val):
    nblk = pl.cdiv(_N, _BN)
    spec_a = pl.BlockSpec((_A, _BN), lambda i: (0, i))
    spec_n = pl.BlockSpec((_BN,), lambda i: (i,))
    return pl.pallas_call(
        _update_body,
        grid=(nblk,),
        in_specs=[spec_a] * 5 + [spec_n] * 3,
        out_specs=spec_n,
        out_shape=jax.ShapeDtypeStruct((_N,), jnp.float32),
    )(eme, ema, ama, uj, uk, u, mask, val)


def kernel(vertices, adjacent_vertex_inds, tensor_field, initial_inds,
           initial_values):
    jkf = (adjacent_vertex_inds[:, :, 1]
           | (adjacent_vertex_inds[:, :, 2] << 16)).T.reshape(-1)
    sf = adjacent_vertex_inds[:, :, 3].T.reshape(-1)
    vx = vertices[:, 0]
    vy = vertices[:, 1]
    m00 = tensor_field[:, 0, 0]
    m01 = tensor_field[:, 0, 1]
    m11 = tensor_field[:, 1, 1]

    xjx, xjy, xkx, xky, g00, g01, g11 = (
        a.reshape(_A, _N)
        for a in _pre_gather(vx, vy, m00, m01, m11, jkf, sf))
    eme, ema, ama = _tc_geom(xjx, xjy, xkx, xky, g00, g01, g11, vx, vy)

    mask = jnp.ones((_N,), jnp.float32).at[initial_inds].set(0.0)
    val = jnp.zeros((_N,), jnp.float32).at[initial_inds].set(initial_values)
    u = jnp.full((_N,), _MAX_VALUE,
                 jnp.float32).at[initial_inds].set(initial_values)

    for _ in range(_ITERS):
        uj, uk = _iter_gather(u, jkf)
        u = _tc_update(eme, ema, ama, uj.reshape(_A, _N), uk.reshape(_A, _N),
                       u, mask, val)
    return u


# speedup vs baseline: 1.0056x; 1.0056x over previous
import functools

import jax
import jax.numpy as jnp
from jax import lax
from jax.experimental import pallas as pl
from jax.experimental.pallas import tpu as pltpu
from jax.experimental.pallas import tpu_sc as plsc

_MAX_VALUE = 1000.0
_ORD = 20.0
_CUT = 0.1
_N = 50000
_S = 100000
_A = 16
_E = _N * _A
_ITERS = 8
_BN = 4096

_NW = 32
_PW = _E // _NW
_CH = 25008
_PSUB = (6256, 6256, 6256, 6240)


def _worker_start(wid):
    return wid * _PW - 8 * (wid % 2)


def _wid():
    return lax.axis_index("s") * 2 + lax.axis_index("c")


def _gather_loop(table_v, idx_v, out_v, n_vec):
    @plsc.parallel_loop(0, n_vec * 16, step=16, unroll=8)
    def _(i):
        idx = idx_v[pl.ds(i, 16)]
        out_v[pl.ds(i, 16)] = plsc.load_gather(table_v, [idx])


_CH3 = _CH // 3


def _iter_gather_body(u_hbm, jk_hbm, uj_hbm, uk_hbm, u_v, idx0, idx1, outj0,
                      outj1, outk0, outk1, u_sem, isem0, isem1, ojsem0,
                      ojsem1, oksem0, oksem1):
    start = _worker_start(_wid())
    idx_bufs = (idx0, idx1)
    outj_bufs = (outj0, outj1)
    outk_bufs = (outk0, outk1)
    isems = (isem0, isem1)
    ojsems = (ojsem0, ojsem1)
    oksems = (oksem0, oksem1)
    u_copy = pltpu.make_async_copy(u_hbm, u_v, u_sem)
    u_copy.start()

    def in_copy(c):
        return pltpu.make_async_copy(
            jk_hbm.at[pl.ds(start + c * _CH3, _CH3)], idx_bufs[c % 2],
            isems[c % 2])

    def out_copies(c):
        return (
            pltpu.make_async_copy(
                outj_bufs[c % 2], uj_hbm.at[pl.ds(start + c * _CH3, _CH3)],
                ojsems[c % 2]),
            pltpu.make_async_copy(
                outk_bufs[c % 2], uk_hbm.at[pl.ds(start + c * _CH3, _CH3)],
                oksems[c % 2]),
        )

    copies_in = [in_copy(c) for c in range(3)]
    copies_out = [out_copies(c) for c in range(3)]
    copies_in[0].start()
    for c in range(3):
        if c + 1 < 3:
            copies_in[c + 1].start()
        copies_in[c].wait()
        if c == 0:
            u_copy.wait()
        if c >= 2:
            for cp in copies_out[c - 2]:
                cp.wait()
        idx_v = idx_bufs[c % 2]
        outj_v = outj_bufs[c % 2]
        outk_v = outk_bufs[c % 2]

        @plsc.parallel_loop(0, _CH3, step=16, unroll=8)
        def _(i):
            jk = idx_v[pl.ds(i, 16)]
            jdx = jnp.bitwise_and(jk, 0xFFFF)
            kdx = lax.shift_right_logical(jk, 16)
            outj_v[pl.ds(i, 16)] = plsc.load_gather(u_v, [jdx])
            outk_v[pl.ds(i, 16)] = plsc.load_gather(u_v, [kdx])

        for cp in copies_out[c]:
            cp.start()
    for c in (1, 2):
        for cp in copies_out[c]:
            cp.wait()


_SC_PARAMS = pltpu.CompilerParams(needs_layout_passes=False)

_iter_gather = functools.partial(
    pl.kernel,
    mesh=plsc.VectorSubcoreMesh(core_axis_name="c", subcore_axis_name="s"),
    compiler_params=_SC_PARAMS,
    out_type=[jax.ShapeDtypeStruct((_E,), jnp.float32)] * 2,
    scratch_types=[
        pltpu.VMEM((_N,), jnp.float32),
        pltpu.VMEM((_CH3,), jnp.int32),
        pltpu.VMEM((_CH3,), jnp.int32),
        pltpu.VMEM((_CH3,), jnp.float32),
        pltpu.VMEM((_CH3,), jnp.float32),
        pltpu.VMEM((_CH3,), jnp.float32),
        pltpu.VMEM((_CH3,), jnp.float32),
        pltpu.SemaphoreType.DMA,
        pltpu.SemaphoreType.DMA,
        pltpu.SemaphoreType.DMA,
        pltpu.SemaphoreType.DMA,
        pltpu.SemaphoreType.DMA,
        pltpu.SemaphoreType.DMA,
        pltpu.SemaphoreType.DMA,
    ],
)(_iter_gather_body)


def _pre_gather_body(vx_hbm, vy_hbm, m00_hbm, m01_hbm, m11_hbm, jk_hbm,
                     sf_hbm, xjx_hbm, xjy_hbm, xkx_hbm, xky_hbm, g00_hbm,
                     g01_hbm, g11_hbm, table_v, idx_v, outa_v, outb_v):
    start = _worker_start(_wid())

    def vertex_chunk(outj_hbm, outk_hbm, off, cnt):
        pltpu.sync_copy(jk_hbm.at[pl.ds(off, cnt)], idx_v.at[pl.ds(0, cnt)])

        @plsc.parallel_loop(0, cnt, step=16, unroll=8)
        def _(i):
            jk = idx_v[pl.ds(i, 16)]
            jdx = jnp.bitwise_and(jk, 0xFFFF)
            kdx = lax.shift_right_logical(jk, 16)
            outa_v[pl.ds(i, 16)] = plsc.load_gather(table_v, [jdx])
            outb_v[pl.ds(i, 16)] = plsc.load_gather(table_v, [kdx])

        pltpu.sync_copy(outa_v.at[pl.ds(0, cnt)], outj_hbm.at[pl.ds(off,
                                                                    cnt)])
        pltpu.sync_copy(outb_v.at[pl.ds(0, cnt)], outk_hbm.at[pl.ds(off,
                                                                    cnt)])

    def sf_chunk(out_hbm, off, cnt):
        pltpu.sync_copy(sf_hbm.at[pl.ds(off, cnt)], idx_v.at[pl.ds(0, cnt)])
        _gather_loop(table_v, idx_v, outa_v, cnt // 16)
        pltpu.sync_copy(outa_v.at[pl.ds(0, cnt)], out_hbm.at[pl.ds(off,
                                                                   cnt)])

    offs = []
    off = 0
    for cnt in _PSUB:
        offs.append((off, cnt))
        off += cnt

    for tbl_hbm, outj_hbm, outk_hbm in ((vx_hbm, xjx_hbm, xkx_hbm),
                                        (vy_hbm, xjy_hbm, xky_hbm)):
        pltpu.sync_copy(tbl_hbm, table_v.at[pl.ds(0, _N)])
        for off, cnt in offs:
            vertex_chunk(outj_hbm, outk_hbm, start + off, cnt)

    for tbl_hbm, out_hbm in ((m00_hbm, g00_hbm), (m01_hbm, g01_hbm),
                             (m11_hbm, g11_hbm)):
        pltpu.sync_copy(tbl_hbm, table_v.at[pl.ds(0, _S)])
        for off, cnt in offs:
            sf_chunk(out_hbm, start + off, cnt)


_pre_gather = functools.partial(
    pl.kernel,
    mesh=plsc.VectorSubcoreMesh(core_axis_name="c", subcore_axis_name="s"),
    compiler_params=_SC_PARAMS,
    out_type=[jax.ShapeDtypeStruct((_E,), jnp.float32)] * 7,
    scratch_types=[
        pltpu.VMEM((_S,), jnp.float32),
        pltpu.VMEM((_PSUB[0],), jnp.int32),
        pltpu.VMEM((_PSUB[0],), jnp.float32),
        pltpu.VMEM((_PSUB[0],), jnp.float32),
    ],
)(_pre_gather_body)


def _softplus(x):
    return jnp.maximum(x, 0.0) + jnp.log1p(jnp.exp(-jnp.abs(x)))


def _drelu(x):
    return _softplus(_ORD * x) / _ORD - _softplus(_ORD * (x - 1.0)) / _ORD


def _geom_body(xjx_ref, xjy_ref, xkx_ref, xky_ref, g00_ref, g01_ref,
               g11_ref, vx_ref, vy_ref, eme_ref, ema_ref, ama_ref):
    xkx = xkx_ref[...]
    xky = xky_ref[...]
    g00 = g00_ref[...]
    g01 = g01_ref[...]
    g11 = g11_ref[...]
    ex = xjx_ref[...] - xkx
    ey = xjy_ref[...] - xky
    ax = vx_ref[...][None, :] - xkx
    ay = vy_ref[...][None, :] - xky
    mex = g00 * ex + g01 * ey
    mey = g01 * ex + g11 * ey
    max_ = g00 * ax + g01 * ay
    may_ = g01 * ax + g11 * ay
    eme_ref[...] = ex * mex + ey * mey
    ema_ref[...] = ex * max_ + ey * may_
    ama_ref[...] = ax * max_ + ay * may_


def _tc_geom(xjx, xjy, xkx, xky, g00, g01, g11, vx, vy):
    nblk = pl.cdiv(_N, _BN)
    spec_a = pl.BlockSpec((_A, _BN), lambda i: (0, i))
    spec_n = pl.BlockSpec((_BN,), lambda i: (i,))
    return pl.pallas_call(
        _geom_body,
        grid=(nblk,),
        in_specs=[spec_a] * 7 + [spec_n] * 2,
        out_specs=[spec_a] * 3,
        out_shape=[jax.ShapeDtypeStruct((_A, _N), jnp.float32)] * 3,
    )(xjx, xjy, xkx, xky, g00, g01, g11, vx, vy)


def _update_body(eme_ref, ema_ref, ama_ref, uj_ref, uk_ref, u_ref, mask_ref,
                 val_ref, out_ref):
    eme = eme_ref[...]
    ema = ema_ref[...]
    ama = ama_ref[...]
    uj = uj_ref[...]
    uk = uk_ref[...]

    delta = uj - uk
    d2 = delta * delta
    qa = eme * (eme - d2)
    qb = 2.0 * ema * (d2 - eme)
    qc = ema * ema - d2 * ama
    disc = qb * qb - 4.0 * qa * qc
    valid = disc >= 0.0
    sq = jnp.sqrt(jnp.maximum(disc, 0.0))
    denom = 2.0 * qa + jnp.where(jnp.abs(qa) < 1e-8, 1e-8, 0.0)
    lam1 = (-qb + sq) / denom
    lam2 = (-qb - sq) / denom
    l1c = _drelu(lam1)
    l2c = _drelu(lam2)

    def ttime(lam):
        dmd = ama + lam * (lam * eme - 2.0 * ema)
        return (1.0 - lam) * uk + lam * uj + jnp.sqrt(
            jnp.clip(dmd, 1e-8, None))

    t1 = jnp.where(valid & (jnp.abs(lam1 - l1c) <= _CUT), ttime(l1c),
                   _MAX_VALUE)
    t2 = jnp.where(valid & (jnp.abs(lam2 - l2c) <= _CUT), ttime(l2c),
                   _MAX_VALUE)
    t0 = uk + jnp.sqrt(jnp.clip(ama, 1e-8, None))
    t3 = uj + jnp.sqrt(jnp.clip(ama - 2.0 * ema + eme, 1e-8, None))
    o = _ORD
    m = jnp.maximum(jnp.maximum(-o * t1, -o * t2),
                    jnp.maximum(-o * t0, -o * t3))
    m2 = jnp.max(m, axis=0)
    mb = m2[None, :]
    ssum = ((jnp.exp(-o * t1 - mb) + jnp.exp(-o * t2 - mb)) +
            (jnp.exp(-o * t0 - mb) + jnp.exp(-o * t3 - mb)))
    s2 = jnp.sum(ssum, axis=0)
    pv = -(m2 + jnp.log(s2)) / o
    unew = jnp.minimum(pv, u_ref[...])
    out_ref[...] = jnp.where(mask_ref[...] > 0.5, unew, val_ref[...])


def _tc_update(eme, ema, ama, uj, uk, u, mask, val):
    nblk = pl.cdiv(_N, _BN)
    spec_a = pl.BlockSpec((_A, _BN), lambda i: (0, i))
    spec_n = pl.BlockSpec((_BN,), lambda i: (i,))
    return pl.pallas_call(
        _update_body,
        grid=(nblk,),
        in_specs=[spec_a] * 5 + [spec_n] * 3,
        out_specs=spec_n,
        out_shape=jax.ShapeDtypeStruct((_N,), jnp.float32),
    )(eme, ema, ama, uj, uk, u, mask, val)


def kernel(vertices, adjacent_vertex_inds, tensor_field, initial_inds,
           initial_values):
    jkf = (adjacent_vertex_inds[:, :, 1]
           | (adjacent_vertex_inds[:, :, 2] << 16)).T.reshape(-1)
    sf = adjacent_vertex_inds[:, :, 3].T.reshape(-1)
    vx = vertices[:, 0]
    vy = vertices[:, 1]
    m00 = tensor_field[:, 0, 0]
    m01 = tensor_field[:, 0, 1]
    m11 = tensor_field[:, 1, 1]

    xjx, xjy, xkx, xky, g00, g01, g11 = (
        a.reshape(_A, _N)
        for a in _pre_gather(vx, vy, m00, m01, m11, jkf, sf))
    eme, ema, ama = _tc_geom(xjx, xjy, xkx, xky, g00, g01, g11, vx, vy)

    mask = jnp.ones((_N,), jnp.float32).at[initial_inds].set(0.0)
    val = jnp.zeros((_N,), jnp.float32).at[initial_inds].set(initial_values)
    u = jnp.full((_N,), _MAX_VALUE,
                 jnp.float32).at[initial_inds].set(initial_values)

    for _ in range(_ITERS):
        uj, uk = _iter_gather(u, jkf)
        u = _tc_update(eme, ema, ama, uj.reshape(_A, _N), uk.reshape(_A, _N),
                       u, mask, val)
    return u
